# Initial kernel scaffold; baseline (speedup 1.0000x reference)
import jax
import jax.numpy as jnp
from jax.experimental import pallas as pl


def kernel(x, edge_index, batch, edge_weight, W1, b1, W2, b2, W3, b3):
    raise NotImplementedError("write your pallas kernel here")


# speedup vs baseline: 7.0504x; 7.0504x over previous
import functools

import jax
import jax.numpy as jnp
from jax import lax
from jax.experimental import pallas as pl
from jax.experimental.pallas import tpu as pltpu
from jax.experimental.pallas import tpu_sc as plsc

N = 100000
NP = 100096
E = 6400000
G = 512
D = 16
NC = 2
NS = 16
HALF = NP // 2
ACCR = HALF + 8
CH = 80
EPT = E // NS
NCHUNK = EPT // CH
RPS = HALF // NS

_mesh = plsc.VectorSubcoreMesh(
    core_axis_name="c", subcore_axis_name="s", num_cores=NC, num_subcores=NS)

_f32 = jnp.float32
_i32 = jnp.int32

_GDN = lax.GatherDimensionNumbers(
    offset_dims=(), collapsed_slice_dims=(0,), start_index_map=(0,))


def _splat(vec, lane):
  idx = jnp.full((16, 1), lane, _i32)
  return lax.gather(vec, idx, _GDN, (1,),
                    mode=lax.GatherScatterMode.PROMISE_IN_BOUNDS)


def _make_edge_pass(with_deg):
  out_type = [jax.ShapeDtypeStruct((NP, D), _f32)]
  if with_deg:
    out_type.append(jax.ShapeDtypeStruct((NP,), _f32))
  scratch = [
      pltpu.VMEM((CH,), _i32),
      pltpu.VMEM((CH,), _i32),
      pltpu.VMEM((CH,), _i32),
      pltpu.VMEM((CH,), _f32),
      pltpu.VMEM((CH, D), _f32),
      pltpu.VMEM((RPS, D), _f32),
      pltpu.VMEM_SHARED((ACCR, D), _f32),
  ]
  if with_deg:
    scratch.append(pltpu.VMEM((RPS,), _f32))
    scratch.append(pltpu.VMEM_SHARED((ACCR,), _f32))
  scratch.append(pltpu.SemaphoreType.DMA)

  @functools.partial(
      pl.kernel, out_type=out_type, mesh=_mesh, scratch_types=scratch,
      compiler_params=pltpu.CompilerParams(use_tc_tiling_on_sc=False))
  def edge_pass(table, srce, dste, ewe, z2d, *rest):
    if with_deg:
      (z1d, a_o, d_o,
       srcv, dstv, dstl, ewv, rows, bounce, accsh, dbounce, degsh, sem) = rest
    else:
      a_o, srcv, dstv, dstl, ewv, rows, bounce, accsh, sem = rest
    c = lax.axis_index("c")
    s = lax.axis_index("s")
    base = c * HALF
    rs = pl.ds(s * RPS, RPS)

    pltpu.sync_copy(z2d, bounce)
    pltpu.sync_copy(bounce, accsh.at[rs])
    if with_deg:
      pltpu.sync_copy(z1d, dbounce)
      pltpu.sync_copy(dbounce, degsh.at[rs])
    plsc.subcore_barrier()

    def body(i, carry):
      off = pl.multiple_of(s * EPT + i * CH, 8)
      pltpu.sync_copy(srce.at[pl.ds(off, CH)], srcv)
      pltpu.sync_copy(dste.at[pl.ds(off, CH)], dstv)
      pltpu.sync_copy(ewe.at[pl.ds(off, CH)], ewv)
      pltpu.async_copy(table.at[srcv], rows, sem).wait()
      for g in range(CH // 16):
        gsl = pl.ds(g * 16, 16)
        dv = dstv[gsl] - base
        dstl[gsl] = jnp.where((dv >= 0) & (dv < HALF), dv, HALF)
        evw = ewv[gsl]
        for r16 in range(16):
          r = g * 16 + r16
          rows[r] = rows[r] * _splat(evw, r16)
      pltpu.sync_copy(rows, accsh.at[dstl], add=True)
      if with_deg:
        pltpu.sync_copy(ewv, degsh.at[dstl], add=True)
      return carry

    lax.fori_loop(0, NCHUNK, body, 0)
    plsc.subcore_barrier()

    out_rs = pl.ds(base + s * RPS, RPS)
    pltpu.sync_copy(accsh.at[rs], bounce)
    pltpu.sync_copy(bounce, a_o.at[out_rs])
    if with_deg:
      pltpu.sync_copy(degsh.at[rs], dbounce)
      pltpu.sync_copy(dbounce, d_o.at[out_rs])

  return edge_pass


_edge_pass_deg = _make_edge_pass(True)
_edge_pass = _make_edge_pass(False)


_pool_scratch = [
    pltpu.VMEM((CH,), _i32),
    pltpu.VMEM((CH, D), _f32),
    pltpu.VMEM((CH, D), _f32),
    pltpu.VMEM((CH, D), _f32),
    pltpu.VMEM((16,), _f32),
    pltpu.VMEM((G // NS, D), _f32),
    pltpu.VMEM_SHARED((G, D), _f32),
    pltpu.SemaphoreType.DMA,
]
_NPCH = N // CH
_NW = NC * NS
_PPT = -(-_NPCH // _NW)
_GRS = G // NS


@functools.partial(
    pl.kernel,
    out_type=[jax.ShapeDtypeStruct((G, D), _f32)] * 2,
    mesh=_mesh, scratch_types=_pool_scratch,
    compiler_params=pltpu.CompilerParams(use_tc_tiling_on_sc=False))
def _sc_pool(a3, t3, b3p, batch, zp, p0_o, p1_o,
             batchv, av, t3v, outv, b3v, pbounce, poolsh, sem):
  c = lax.axis_index("c")
  s = lax.axis_index("s")
  w = s * NC + c
  gs = pl.ds(s * _GRS, _GRS)
  pltpu.sync_copy(b3p, b3v)
  pltpu.sync_copy(zp, pbounce)
  pltpu.sync_copy(pbounce, poolsh.at[gs])
  plsc.subcore_barrier()

  def body(j, carry):
    cid = j * _NW + w
    @pl.when(cid < _NPCH)
    def _():
      off = pl.multiple_of(cid * CH, 8)
      pltpu.sync_copy(batch.at[pl.ds(off, CH)], batchv)
      pltpu.sync_copy(a3.at[pl.ds(off, CH)], av)
      pltpu.sync_copy(t3.at[pl.ds(off, CH)], t3v)
      bvec = b3v[...]
      for r in range(CH):
        outv[r] = jnp.maximum(av[r] + t3v[r] + bvec, 0.0)
      pltpu.sync_copy(outv, poolsh.at[batchv], add=True)
    return carry

  lax.fori_loop(0, _PPT, body, 0)
  plsc.subcore_barrier()
  pltpu.sync_copy(poolsh.at[gs], pbounce)
  @pl.when(c == 0)
  def _():
    pltpu.sync_copy(pbounce, p0_o.at[gs])
  @pl.when(c == 1)
  def _():
    pltpu.sync_copy(pbounce, p1_o.at[gs])


_BR = 3128


def _row_spec():
  return pl.BlockSpec((_BR, D), lambda i: (i, 0))


def _vec_spec():
  return pl.BlockSpec((_BR, 1), lambda i: (i, 0))


def _full_spec(shape):
  return pl.BlockSpec(shape, lambda i: tuple(0 for _ in shape))


def _tc_table1_body(x_ref, w_ref, o_ref):
  x = x_ref[...]
  t = x[:, 0:1] * w_ref[0:1, :]
  for k in range(1, 3):
    t += x[:, k:k + 1] * w_ref[k:k + 1, :]
  o_ref[...] = t


def _tc_table1(xp, w1p):
  return pl.pallas_call(
      _tc_table1_body,
      grid=(NP // _BR,),
      in_specs=[pl.BlockSpec((_BR, 8), lambda i: (i, 0)), _full_spec((8, D))],
      out_specs=_row_spec(),
      out_shape=jax.ShapeDtypeStruct((NP, D), _f32),
  )(xp, w1p)


def _tc_dense1_body(a_ref, t1_ref, dg_ref, b1_ref, w2_ref, t2_ref, dis_ref):
  h1 = jnp.maximum(a_ref[...] + t1_ref[...] + b1_ref[0:1, :], 0.0)
  deg = dg_ref[...]
  dis = jnp.where(deg > 0, lax.rsqrt(jnp.where(deg > 0, deg, 1.0)), 0.0)
  t2 = h1[:, 0:1] * w2_ref[0:1, :]
  for k in range(1, 7):
    t2 += h1[:, k:k + 1] * w2_ref[k:k + 1, :]
  t2_ref[...] = dis * t2
  dis_ref[...] = dis


def _tc_dense1(a1, t1, dg, b1p, w2p):
  return pl.pallas_call(
      _tc_dense1_body,
      grid=(NP // _BR,),
      in_specs=[_row_spec(), _row_spec(), _vec_spec(),
                _full_spec((8, D)), _full_spec((8, D))],
      out_specs=[_row_spec(), _vec_spec()],
      out_shape=[jax.ShapeDtypeStruct((NP, D), _f32),
                 jax.ShapeDtypeStruct((NP, 1), _f32)],
  )(a1, t1, dg, b1p, w2p)


def _tc_dense2_body(a_ref, dis_ref, b2_ref, w3_ref, t3_ref):
  out2 = jnp.maximum(dis_ref[...] * a_ref[...] + b2_ref[0:1, :], 0.0)
  t3 = out2[:, 0:1] * w3_ref[0:1, :]
  for k in range(1, 9):
    t3 += out2[:, k:k + 1] * w3_ref[k:k + 1, :]
  t3_ref[...] = t3


def _tc_dense2(a2, dis, b2p, w3p):
  return pl.pallas_call(
      _tc_dense2_body,
      grid=(NP // _BR,),
      in_specs=[_row_spec(), _vec_spec(),
                _full_spec((8, D)), _full_spec((16, D))],
      out_specs=_row_spec(),
      out_shape=jax.ShapeDtypeStruct((NP, D), _f32),
  )(a2, dis, b2p, w3p)


def _tc_softmax_body(p0_ref, p1_ref, o_ref):
  p = p0_ref[...] + p1_ref[...]
  q = p[:, :6]
  m = jnp.max(q, axis=1, keepdims=True)
  e = jnp.exp(q - m)
  ssum = jnp.sum(e, axis=1, keepdims=True)
  o_ref[...] = (q - m) - jnp.log(ssum)


def _tc_softmax(p0, p1):
  return pl.pallas_call(
      _tc_softmax_body,
      out_shape=jax.ShapeDtypeStruct((G, 6), _f32),
  )(p0, p1)


def kernel(x, edge_index, batch, edge_weight, W1, b1, W2, b2, W3, b3):
  src = edge_index[0].astype(_i32)
  dst = edge_index[1].astype(_i32)
  batchi = batch.astype(_i32)
  ew = edge_weight.astype(_f32)

  xp = jnp.zeros((NP, 8), _f32).at[:N, :3].set(x)
  w1p = jnp.zeros((8, D), _f32).at[:3, :7].set(W1)
  b1p = jnp.zeros((8, D), _f32).at[0, :7].set(b1)
  w2p = jnp.zeros((8, D), _f32).at[:7, :9].set(W2)
  b2p = jnp.zeros((8, D), _f32).at[0, :9].set(b2)
  w3p = jnp.zeros((16, D), _f32).at[:9, :6].set(W3)
  b3p = jnp.zeros((16,), _f32).at[:6].set(b3)
  z2d = jnp.zeros((RPS, D), _f32)
  z1d = jnp.zeros((RPS,), _f32)
  zp = jnp.zeros((_GRS, D), _f32)

  t1 = _tc_table1(xp, w1p)
  a1, dg = _edge_pass_deg(t1, src, dst, ew, z2d, z1d)
  t2, dis = _tc_dense1(a1, t1, dg.reshape(NP, 1), b1p, w2p)
  (a2,) = _edge_pass(t2, src, dst, ew, z2d)
  t3 = _tc_dense2(a2, dis, b2p, w3p)
  (a3,) = _edge_pass(t3, src, dst, ew, z2d)
  p0, p1 = _sc_pool(a3, t3, b3p, batchi, zp)
  return _tc_softmax(p0, p1)


# speedup vs baseline: 19.1300x; 2.7133x over previous
import functools

import jax
import jax.numpy as jnp
from jax import lax
from jax.experimental import pallas as pl
from jax.experimental.pallas import tpu as pltpu
from jax.experimental.pallas import tpu_sc as plsc

N = 100000
NP = 100096
E = 6400000
G = 512
D = 16
NC = 2
NS = 16
HALF = NP // 2
ACCR = HALF + 8
CH = 128
EPT = E // NS
NCHUNK = EPT // CH
PAIRS = (NCHUNK - 1) // 2
RPS = HALF // NS
PCH = 80

_mesh = plsc.VectorSubcoreMesh(
    core_axis_name="c", subcore_axis_name="s", num_cores=NC, num_subcores=NS)

_f32 = jnp.float32
_i32 = jnp.int32

_GDN = lax.GatherDimensionNumbers(
    offset_dims=(), collapsed_slice_dims=(0,), start_index_map=(0,))


def _splat(vec, lane):
  idx = jnp.full((16, 1), lane, _i32)
  return lax.gather(vec, idx, _GDN, (1,),
                    mode=lax.GatherScatterMode.PROMISE_IN_BOUNDS)


def _make_edge_pass(with_deg):
  out_type = [jax.ShapeDtypeStruct((NP, D), _f32)]
  if with_deg:
    out_type.append(jax.ShapeDtypeStruct((NP,), _f32))
  nbuf = 2
  scratch = [
      [pltpu.VMEM((CH,), _i32)] * nbuf,
      [pltpu.VMEM((CH,), _i32)] * nbuf,
      [pltpu.VMEM((CH,), _i32)] * nbuf,
      [pltpu.VMEM((CH,), _f32)] * nbuf,
      [pltpu.VMEM((CH, D), _f32)] * nbuf,
      pltpu.VMEM((RPS, D), _f32),
      pltpu.VMEM_SHARED((ACCR, D), _f32),
      [pltpu.SemaphoreType.DMA] * nbuf,
      [pltpu.SemaphoreType.DMA] * nbuf,
      [pltpu.SemaphoreType.DMA] * nbuf,
  ]
  if with_deg:
    scratch.append([pltpu.VMEM((CH,), _f32)] * nbuf)
    scratch.append(pltpu.VMEM((RPS,), _f32))
    scratch.append(pltpu.VMEM_SHARED((ACCR,), _f32))
    scratch.append([pltpu.SemaphoreType.DMA] * nbuf)

  @functools.partial(
      pl.kernel, out_type=out_type, mesh=_mesh, scratch_types=scratch,
      compiler_params=pltpu.CompilerParams(use_tc_tiling_on_sc=False))
  def edge_pass(table, srce, dste, ewe, z2d, *rest):
    if with_deg:
      (z1d, a_o, d_o, srcv, dstv, dstl, ewv, rows, bounce, accsh,
       lsem, gsem, ssem, ewd, dbounce, degsh, dsem) = rest
    else:
      (a_o, srcv, dstv, dstl, ewv, rows, bounce, accsh,
       lsem, gsem, ssem) = rest
    c = lax.axis_index("c")
    s = lax.axis_index("s")
    base = c * HALF
    rs = pl.ds(s * RPS, RPS)

    pltpu.sync_copy(z2d, bounce)
    pltpu.sync_copy(bounce, accsh.at[rs])
    if with_deg:
      pltpu.sync_copy(z1d, dbounce)
      pltpu.sync_copy(dbounce, degsh.at[rs])
    plsc.subcore_barrier()

    def lin_issue(cid, p):
      off = pl.multiple_of(
          s * EPT + jnp.where(cid < NCHUNK, cid, 0) * CH, 8)
      pltpu.async_copy(srce.at[pl.ds(off, CH)], srcv[p], lsem[p])
      pltpu.async_copy(dste.at[pl.ds(off, CH)], dstv[p], lsem[p])
      pltpu.async_copy(ewe.at[pl.ds(off, CH)], ewv[p], lsem[p])

    def lin_wait(p):
      dummy = pl.ds(0, CH)
      pltpu.make_async_copy(srce.at[dummy], srcv[p], lsem[p]).wait()
      pltpu.make_async_copy(dste.at[dummy], dstv[p], lsem[p]).wait()
      pltpu.make_async_copy(ewe.at[dummy], ewv[p], lsem[p]).wait()

    def gat_issue(p):
      pltpu.async_copy(table.at[srcv[p]], rows[p], gsem[p])

    def gat_wait(p):
      pltpu.make_async_copy(table.at[srcv[p]], rows[p], gsem[p]).wait()

    def sca_issue(p):
      pltpu.async_copy(rows[p], accsh.at[dstl[p]], ssem[p], add=True)
      if with_deg:
        pltpu.async_copy(ewd[p], degsh.at[dstl[p]], dsem[p], add=True)

    def sca_wait(p):
      pltpu.make_async_copy(rows[p], accsh.at[dstl[p]], ssem[p]).wait()
      if with_deg:
        pltpu.make_async_copy(ewd[p], degsh.at[dstl[p]], dsem[p]).wait()

    def compute(p):
      for g in range(CH // 16):
        gsl = pl.ds(g * 16, 16)
        dv = dstv[p][gsl] - base
        dstl[p][gsl] = jnp.where((dv >= 0) & (dv < HALF), dv, HALF)
        evw = ewv[p][gsl]
        if with_deg:
          ewd[p][gsl] = evw
        for r16 in range(16):
          r = g * 16 + r16
          rows[p][r] = rows[p][r] * _splat(evw, r16)

    lin_issue(0, 0)
    lin_issue(1, 1)
    for g in range(CH // 16):
      dstl[1][pl.ds(g * 16, 16)] = jnp.full((16,), HALF, _i32)
    sca_issue(1)
    lin_wait(0)
    gat_issue(0)

    def body(g, carry):
      a = 2 * g
      gat_wait(0)
      lin_wait(1)
      sca_wait(1)
      gat_issue(1)
      compute(0)
      sca_issue(0)
      lin_issue(a + 2, 0)
      gat_wait(1)
      lin_wait(0)
      sca_wait(0)
      gat_issue(0)
      compute(1)
      sca_issue(1)
      lin_issue(a + 3, 1)
      return carry

    lax.fori_loop(0, PAIRS, body, 0)

    gat_wait(0)
    lin_wait(1)
    sca_wait(1)
    compute(0)
    sca_issue(0)
    sca_wait(0)
    plsc.subcore_barrier()

    out_rs = pl.ds(base + s * RPS, RPS)
    pltpu.sync_copy(accsh.at[rs], bounce)
    pltpu.sync_copy(bounce, a_o.at[out_rs])
    if with_deg:
      pltpu.sync_copy(degsh.at[rs], dbounce)
      pltpu.sync_copy(dbounce, d_o.at[out_rs])

  return edge_pass


_edge_pass_deg = _make_edge_pass(True)
_edge_pass = _make_edge_pass(False)


_pool_scratch = [
    pltpu.VMEM((PCH,), _i32),
    pltpu.VMEM((PCH, D), _f32),
    pltpu.VMEM((PCH, D), _f32),
    pltpu.VMEM((PCH, D), _f32),
    pltpu.VMEM((16,), _f32),
    pltpu.VMEM((G // NS, D), _f32),
    pltpu.VMEM_SHARED((G, D), _f32),
    pltpu.SemaphoreType.DMA,
]
_NPCH = N // PCH
_NW = NC * NS
_PPT = -(-_NPCH // _NW)
_GRS = G // NS


@functools.partial(
    pl.kernel,
    out_type=[jax.ShapeDtypeStruct((G, D), _f32)] * 2,
    mesh=_mesh, scratch_types=_pool_scratch,
    compiler_params=pltpu.CompilerParams(use_tc_tiling_on_sc=False))
def _sc_pool(a3, t3, b3p, batch, zp, p0_o, p1_o,
             batchv, av, t3v, outv, b3v, pbounce, poolsh, sem):
  c = lax.axis_index("c")
  s = lax.axis_index("s")
  w = s * NC + c
  gs = pl.ds(s * _GRS, _GRS)
  pltpu.sync_copy(b3p, b3v)
  pltpu.sync_copy(zp, pbounce)
  pltpu.sync_copy(pbounce, poolsh.at[gs])
  plsc.subcore_barrier()

  def body(j, carry):
    cid = j * _NW + w
    @pl.when(cid < _NPCH)
    def _():
      off = pl.multiple_of(cid * PCH, 8)
      pltpu.sync_copy(batch.at[pl.ds(off, PCH)], batchv)
      pltpu.sync_copy(a3.at[pl.ds(off, PCH)], av)
      pltpu.sync_copy(t3.at[pl.ds(off, PCH)], t3v)
      bvec = b3v[...]
      for r in range(PCH):
        outv[r] = jnp.maximum(av[r] + t3v[r] + bvec, 0.0)
      pltpu.sync_copy(outv, poolsh.at[batchv], add=True)
    return carry

  lax.fori_loop(0, _PPT, body, 0)
  plsc.subcore_barrier()
  pltpu.sync_copy(poolsh.at[gs], pbounce)
  @pl.when(c == 0)
  def _():
    pltpu.sync_copy(pbounce, p0_o.at[gs])
  @pl.when(c == 1)
  def _():
    pltpu.sync_copy(pbounce, p1_o.at[gs])


_BR = 3128


def _row_spec():
  return pl.BlockSpec((_BR, D), lambda i: (i, 0))


def _vec_spec():
  return pl.BlockSpec((_BR, 1), lambda i: (i, 0))


def _full_spec(shape):
  return pl.BlockSpec(shape, lambda i: tuple(0 for _ in shape))


def _bf16(v):
  return v.astype(jnp.bfloat16).astype(_f32)


def _tc_table1_body(x_ref, w_ref, o_ref):
  x = _bf16(x_ref[...])
  w_ref = _bf16(w_ref[...])
  t = x[:, 0:1] * w_ref[0:1, :]
  for k in range(1, 3):
    t += x[:, k:k + 1] * w_ref[k:k + 1, :]
  o_ref[...] = t


def _tc_table1(xp, w1p):
  return pl.pallas_call(
      _tc_table1_body,
      grid=(NP // _BR,),
      in_specs=[pl.BlockSpec((_BR, 8), lambda i: (i, 0)), _full_spec((8, D))],
      out_specs=_row_spec(),
      out_shape=jax.ShapeDtypeStruct((NP, D), _f32),
  )(xp, w1p)


def _tc_dense1_body(a_ref, t1_ref, dg_ref, b1_ref, w2_ref, t2_ref, dis_ref):
  h1 = jnp.maximum(a_ref[...] + t1_ref[...] + b1_ref[0:1, :], 0.0)
  deg = dg_ref[...]
  dis = jnp.where(deg > 0, lax.rsqrt(jnp.where(deg > 0, deg, 1.0)), 0.0)
  h1b = _bf16(h1)
  w2b = _bf16(w2_ref[...])
  t2 = h1b[:, 0:1] * w2b[0:1, :]
  for k in range(1, 7):
    t2 += h1b[:, k:k + 1] * w2b[k:k + 1, :]
  t2_ref[...] = dis * t2
  dis_ref[...] = dis


def _tc_dense1(a1, t1, dg, b1p, w2p):
  return pl.pallas_call(
      _tc_dense1_body,
      grid=(NP // _BR,),
      in_specs=[_row_spec(), _row_spec(), _vec_spec(),
                _full_spec((8, D)), _full_spec((8, D))],
      out_specs=[_row_spec(), _vec_spec()],
      out_shape=[jax.ShapeDtypeStruct((NP, D), _f32),
                 jax.ShapeDtypeStruct((NP, 1), _f32)],
  )(a1, t1, dg, b1p, w2p)


def _tc_dense2_body(a_ref, dis_ref, b2_ref, w3_ref, t2_ref, t3_ref):
  del t2_ref
  out2 = jnp.maximum(dis_ref[...] * a_ref[...] + b2_ref[0:1, :], 0.0)
  o2b = _bf16(out2)
  w3b = _bf16(w3_ref[...])
  t3 = o2b[:, 0:1] * w3b[0:1, :]
  for k in range(1, 9):
    t3 += o2b[:, k:k + 1] * w3b[k:k + 1, :]
  t3_ref[...] = t3


def _tc_dense2(a2, dis, b2p, w3p, t2):
  return pl.pallas_call(
      _tc_dense2_body,
      grid=(NP // _BR,),
      in_specs=[_row_spec(), _vec_spec(),
                _full_spec((8, D)), _full_spec((16, D)), _row_spec()],
      out_specs=_row_spec(),
      out_shape=jax.ShapeDtypeStruct((NP, D), _f32),
  )(a2, dis, b2p, w3p, t2)


def _tc_softmax_body(p0_ref, p1_ref, o_ref):
  p = p0_ref[...] + p1_ref[...]
  q = p[:, :6]
  m = jnp.max(q, axis=1, keepdims=True)
  e = jnp.exp(q - m)
  ssum = jnp.sum(e, axis=1, keepdims=True)
  o_ref[...] = (q - m) - jnp.log(ssum)


def _tc_softmax(p0, p1):
  return pl.pallas_call(
      _tc_softmax_body,
      out_shape=jax.ShapeDtypeStruct((G, 6), _f32),
  )(p0, p1)


def kernel(x, edge_index, batch, edge_weight, W1, b1, W2, b2, W3, b3):
  src = edge_index[0].astype(_i32)
  dst = edge_index[1].astype(_i32)
  batchi = batch.astype(_i32)
  ew = edge_weight.astype(_f32)

  xp = jnp.zeros((NP, 8), _f32).at[:N, :3].set(x)
  w1p = jnp.zeros((8, D), _f32).at[:3, :7].set(W1)
  b1p = jnp.zeros((8, D), _f32).at[0, :7].set(b1)
  w2p = jnp.zeros((8, D), _f32).at[:7, :9].set(W2)
  b2p = jnp.zeros((8, D), _f32).at[0, :9].set(b2)
  w3p = jnp.zeros((16, D), _f32).at[:9, :6].set(W3)
  b3p = jnp.zeros((16,), _f32).at[:6].set(b3)
  z2d = jnp.zeros((RPS, D), _f32)
  z1d = jnp.zeros((RPS,), _f32)
  zp = jnp.zeros((_GRS, D), _f32)

  t1 = _tc_table1(xp, w1p)
  a1, dg = _edge_pass_deg(t1, src, dst, ew, z2d, z1d)
  t2, dis = _tc_dense1(a1, t1, dg.reshape(NP, 1), b1p, w2p)
  (a2,) = _edge_pass(t2, src, dst, ew, z2d)
  t3 = _tc_dense2(a2, dis, b2p, w3p, t2)
  (a3,) = _edge_pass(t3, src, dst, ew, z2d)
  p0, p1 = _sc_pool(a3, t3, b3p, batchi, zp)
  return _tc_softmax(p0, p1)
